# Initial kernel scaffold; baseline (speedup 1.0000x reference)
import jax
import jax.numpy as jnp
from jax.experimental import pallas as pl


def kernel(batched_tokens, table):
    raise NotImplementedError("write your pallas kernel here")


# speedup vs baseline: 1.3997x; 1.3997x over previous
import functools
import math

import jax
import jax.numpy as jnp
from jax import lax
from jax.experimental import pallas as pl
from jax.experimental.pallas import tpu as pltpu
from jax.experimental.pallas import tpu_sc as plsc

_LANES = 16


@functools.cache
def _build(N, V, D, interpret):
    try:
        info = plsc.get_sparse_core_info()
        NC, NS = info.num_cores, info.num_subcores
    except ValueError:
        NC, NS = 2, 16
    NW = NC * NS
    assert N % NW == 0
    b_per_w = N // NW
    C = 1024
    assert b_per_w % C == 0
    nchunks = b_per_w // C
    scale = math.sqrt(D)
    mesh = plsc.VectorSubcoreMesh(
        core_axis_name="c", subcore_axis_name="s", num_cores=NC, num_subcores=NS
    )

    @functools.partial(
        pl.kernel,
        out_type=jax.ShapeDtypeStruct((N, D), jnp.float32),
        mesh=mesh,
        scratch_types=[
            pltpu.VMEM((C,), jnp.int32),
            pltpu.VMEM((C, D), jnp.float32),
            pltpu.SemaphoreType.DMA,
        ],
        compiler_params=pltpu.CompilerParams(use_tc_tiling_on_sc=False),
        interpret=interpret,
    )
    def emb(tok_hbm, table_hbm, out_hbm, idx_v, rows_v, sem):
        wid = lax.axis_index("s") * NC + lax.axis_index("c")
        base = wid * b_per_w

        @pl.loop(0, nchunks)
        def chunk(c):
            off = base + c * C
            pltpu.sync_copy(tok_hbm.at[pl.ds(off, C)], idx_v)
            pltpu.async_copy(table_hbm.at[idx_v], rows_v, sem).wait()

            @pl.loop(0, C, unroll=8)
            def scale_row(i):
                for h in range(D // _LANES):
                    s = pl.ds(h * _LANES, _LANES)
                    rows_v[i, s] = rows_v[i, s] * scale

            pltpu.sync_copy(rows_v, out_hbm.at[pl.ds(off, C)])

    return emb


def kernel(batched_tokens, table):
    B0, S = batched_tokens.shape
    V, D = table.shape
    N = B0 * S
    tokens = batched_tokens.reshape(N).astype(jnp.int32)
    out = _build(N, V, D, False)(tokens, table)
    return out.reshape(B0, S, D)


# speedup vs baseline: 1.4794x; 1.0570x over previous
import functools
import math

import jax
import jax.numpy as jnp
from jax import lax
from jax.experimental import pallas as pl
from jax.experimental.pallas import tpu as pltpu
from jax.experimental.pallas import tpu_sc as plsc

_LANES = 16


@functools.cache
def _build(N, V, D, interpret):
    try:
        info = plsc.get_sparse_core_info()
        NC, NS = info.num_cores, info.num_subcores
    except ValueError:
        NC, NS = 2, 16
    NW = NC * NS
    assert N % NW == 0
    b_per_w = N // NW
    NBUF = 4
    C = 800
    assert b_per_w % (C * NBUF) == 0
    nchunks = b_per_w // C
    nsteps = nchunks // NBUF
    assert nchunks >= 2 * NBUF
    scale = math.sqrt(D)
    mesh = plsc.VectorSubcoreMesh(
        core_axis_name="c", subcore_axis_name="s", num_cores=NC, num_subcores=NS
    )

    @functools.partial(
        pl.kernel,
        out_type=jax.ShapeDtypeStruct((N, D), jnp.float32),
        mesh=mesh,
        scratch_types=[
            pltpu.VMEM((NBUF, C), jnp.int32),
            pltpu.VMEM((NBUF, C, D), jnp.float32),
            pltpu.SemaphoreType.DMA((NBUF,)),
            pltpu.SemaphoreType.DMA((NBUF,)),
            pltpu.SemaphoreType.DMA((NBUF,)),
        ],
        compiler_params=pltpu.CompilerParams(use_tc_tiling_on_sc=False),
        interpret=interpret,
    )
    def emb(tok_hbm, table_hbm, out_hbm, idx_v, rows_v, isem, gsem, osem):
        wid = lax.axis_index("s") * NC + lax.axis_index("c")
        base = wid * b_per_w

        def idx_start(c, b):
            pltpu.async_copy(
                tok_hbm.at[pl.ds(base + c * C, C)], idx_v.at[b], isem.at[b]
            )

        def idx_wait(b):
            pltpu.make_async_copy(
                tok_hbm.at[pl.ds(0, C)], idx_v.at[b], isem.at[b]
            ).wait()

        def gather_start(c, b):
            pltpu.async_copy(table_hbm.at[idx_v.at[b]], rows_v.at[b], gsem.at[b])

        def gather_wait(b):
            pltpu.make_async_copy(
                table_hbm.at[idx_v.at[b]], rows_v.at[b], gsem.at[b]
            ).wait()

        def scatter_start(c, b):
            pltpu.async_copy(
                rows_v.at[b], out_hbm.at[pl.ds(base + c * C, C)], osem.at[b]
            )

        def scatter_wait(b):
            pltpu.make_async_copy(
                rows_v.at[b], out_hbm.at[pl.ds(0, C)], osem.at[b]
            ).wait()

        def scale_rows(b):
            @pl.loop(0, C, unroll=8)
            def scale_row(i):
                for h in range(D // _LANES):
                    s = pl.ds(h * _LANES, _LANES)
                    rows_v[b, i, s] = rows_v[b, i, s] * scale

        for b in range(NBUF):
            idx_start(b, b)
        for b in range(2):
            idx_wait(b)
            gather_start(b, b)

        def step(c, b, static_c=None):
            gather_wait(b)
            scale_rows(b)
            scatter_start(c, b)
            b2 = (b + 2) % NBUF

            def issue_idx():
                idx_start(c + NBUF, b)

            def issue_gather(need_osem):
                idx_wait(b2)
                if need_osem:
                    scatter_wait(b2)
                gather_start(c + 2, b2)

            if static_c is not None:
                if static_c + NBUF < nchunks:
                    issue_idx()
                if static_c + 2 < nchunks:
                    issue_gather(static_c + 2 >= NBUF)
            else:
                pl.when(c + NBUF < nchunks)(issue_idx)
                pl.when(c + 2 < nchunks)(lambda: issue_gather(True))

        for b in range(NBUF):
            step(b, b, static_c=b)

        @pl.loop(1, nsteps)
        def outer(g):
            for b in range(NBUF):
                step(g * NBUF + b, b)

        for b in range(NBUF):
            scatter_wait(b)

    return emb


def kernel(batched_tokens, table):
    B0, S = batched_tokens.shape
    V, D = table.shape
    N = B0 * S
    tokens = batched_tokens.reshape(N).astype(jnp.int32)
    out = _build(N, V, D, False)(tokens, table)
    return out.reshape(B0, S, D)
